# baseline (speedup 1.0000x reference)
import functools
import jax
import jax.numpy as jnp
from jax import lax
from jax.experimental import pallas as pl
from jax.experimental.pallas import tpu as pltpu
from jax.experimental.pallas import tpu_sc as plsc

N = 5000
D = 128
E = 320000
EROWS = E // 128
NWORK = 32
EROWS_PAD = 2560
ROWS_W = EROWS_PAD // NWORK
BLK = 2
IDX_CHUNK = 40
EA_CHUNK = 16
TILE_ROWS = 320
ACC_ROWS = 16 * TILE_ROWS
JUNK_ROW = ACC_ROWS - 1


def _edge_body(table, gidx, sidx, ea, w2, out,
               gidx_v, sidx_v, ea_v, w2_v, rows_v, acc, sem, sem2):
    c = lax.axis_index("c")
    s = lax.axis_index("s")
    wid = s * 2 + c
    base = wid * ROWS_W

    pltpu.sync_copy(w2, w2_v)
    w2v = [[w2_v[k, pl.ds(d * 16, 16)] for d in range(8)] for k in range(4)]

    pltpu.sync_copy(gidx.at[pl.ds(base, IDX_CHUNK)], gidx_v)
    pltpu.sync_copy(sidx.at[pl.ds(base, IDX_CHUNK)], sidx_v)
    pltpu.sync_copy(ea.at[pl.ds(base, EA_CHUNK)], ea_v)

    zrow = jnp.zeros((16,), jnp.float32)

    def zero_body(r, carry):
        for d in range(8):
            rows_v[r, pl.ds(d * 16, 16)] = zrow
        return carry

    lax.fori_loop(0, 128, zero_body, 0)
    base_acc = s * TILE_ROWS
    z128 = rows_v.at[pl.ds(0, 128)]
    pltpu.sync_copy(z128, acc.at[pl.ds(base_acc, 128)])
    pltpu.sync_copy(z128, acc.at[pl.ds(base_acc + 128, 128)])
    pltpu.sync_copy(rows_v.at[pl.ds(0, 64)], acc.at[pl.ds(base_acc + 256, 64)])
    plsc.subcore_barrier()

    dnums = lax.GatherDimensionNumbers(
        offset_dims=(), collapsed_slice_dims=(0,), start_index_map=(0,))

    def compute(ei, br):
        def grp(g, carry2):
            eak = [ea_v[ei, k, pl.ds(g * 16, 16)] for k in range(4)]
            for j in range(16):
                jidx = jnp.full((16, 1), j, jnp.int32)
                b = [lax.gather(eak[k], jidx, dnums, slice_sizes=(1,),
                                mode=lax.GatherScatterMode.PROMISE_IN_BOUNDS)
                     for k in range(4)]
                e = br * 128 + g * 16 + j
                for d in range(8):
                    r = rows_v[e, pl.ds(d * 16, 16)]
                    r = (r + b[0] * w2v[0][d] + b[1] * w2v[1][d]
                         + b[2] * w2v[2][d] + b[3] * w2v[3][d])
                    rows_v[e, pl.ds(d * 16, 16)] = jnp.maximum(r, 0.0)
            return carry2

        lax.fori_loop(0, 8, grp, 0)

    def group(g, carry):
        r0 = BLK * g
        ic = lax.div(g, IDX_CHUNK // BLK)
        ec = lax.div(g, EA_CHUNK // BLK)

        @pl.when(jnp.logical_and(g > 0, lax.rem(g, IDX_CHUNK // BLK) == 0))
        def _():
            pltpu.sync_copy(gidx.at[pl.ds(base + ic * IDX_CHUNK, IDX_CHUNK)],
                            gidx_v)
            pltpu.sync_copy(sidx.at[pl.ds(base + ic * IDX_CHUNK, IDX_CHUNK)],
                            sidx_v)

        @pl.when(jnp.logical_and(g > 0, lax.rem(g, EA_CHUNK // BLK) == 0))
        def _():
            pltpu.sync_copy(ea.at[pl.ds(base + ec * EA_CHUNK, EA_CHUNK)], ea_v)

        lr = lax.rem(r0, IDX_CHUNK)
        for br in range(BLK):
            pltpu.async_copy(table.at[gidx_v.at[lr + br]],
                             rows_v.at[pl.ds(br * 128, 128)], sem)
        for br in range(BLK):
            pltpu.make_async_copy(table.at[gidx_v.at[lr + br]],
                                  rows_v.at[pl.ds(br * 128, 128)], sem).wait()
        for br in range(BLK):
            compute(lax.rem(r0 + br, EA_CHUNK), br)
        for br in range(BLK):
            pltpu.async_copy(rows_v.at[pl.ds(br * 128, 128)],
                             acc.at[sidx_v.at[lr + br]], add=True, sem=sem2)
        for br in range(BLK):
            pltpu.make_async_copy(rows_v.at[pl.ds(br * 128, 128)],
                                  acc.at[sidx_v.at[lr + br]], sem2).wait()
        return carry

    lax.fori_loop(0, ROWS_W // BLK, group, 0)
    plsc.subcore_barrier()

    for off, nr in ((0, 128), (128, 128), (256, 64)):
        pltpu.sync_copy(acc.at[pl.ds(base_acc + off, nr)],
                        rows_v.at[pl.ds(0, nr)])
        pltpu.sync_copy(rows_v.at[pl.ds(0, nr)],
                        out.at[c, pl.ds(base_acc + off, nr)])


def _edge_pass(table, gidx, sidx, ea, w2):
    mesh = plsc.VectorSubcoreMesh(core_axis_name="c", subcore_axis_name="s")
    f = pl.kernel(
        _edge_body,
        mesh=mesh,
        out_type=jax.ShapeDtypeStruct((2, ACC_ROWS, D), jnp.float32),
        scratch_types=[
            pltpu.VMEM((IDX_CHUNK, 128), jnp.int32),
            pltpu.VMEM((IDX_CHUNK, 128), jnp.int32),
            pltpu.VMEM((EA_CHUNK, 4, 128), jnp.float32),
            pltpu.VMEM((4, 128), jnp.float32),
            pltpu.VMEM((BLK * 128, 128), jnp.float32),
            pltpu.VMEM_SHARED((ACC_ROWS, 128), jnp.float32),
            pltpu.SemaphoreType.DMA,
            pltpu.SemaphoreType.DMA,
        ],
    )
    return f(table, gidx, sidx, ea, w2)


def _dot(a, b):
    return jnp.dot(a, b, preferred_element_type=jnp.float32)


def _emb_body(xc, xv, wc, wv, w0, w1, hc_o, hv_o, t0_o, t1_o):
    hc = jnp.maximum(_dot(xc[:], wc[:]), 0.0)
    hv = jnp.maximum(_dot(xv[:], wv[:]), 0.0)
    hc_o[:] = hc
    hv_o[:] = hv
    t0_o[:] = _dot(hv, w0[:])
    t1_o[:] = _dot(hv, w1[:])


def _emb(xc, xv, wc, wv, w0, w1):
    sds = jax.ShapeDtypeStruct((N, D), jnp.float32)
    return pl.pallas_call(
        _emb_body, out_shape=(sds, sds, sds, sds))(xc, xv, wc, wv, w0, w1)


def _up_t_body(h, parts, wa, wb, wn, h_o, t_o):
    agg = parts[0, :N, :] + parts[1, :N, :]
    hn = jnp.maximum(_dot(h[:], wa[:]) + _dot(agg, wb[:]), 0.0)
    h_o[:] = hn
    t_o[:] = _dot(hn, wn[:])


def _up_t(h, parts, wa, wb, wn):
    sds = jax.ShapeDtypeStruct((N, D), jnp.float32)
    return pl.pallas_call(
        _up_t_body, out_shape=(sds, sds))(h, parts, wa, wb, wn)


def _up_body(h, parts, wa, wb, h_o):
    agg = parts[0, :N, :] + parts[1, :N, :]
    h_o[:] = jnp.maximum(_dot(h[:], wa[:]) + _dot(agg, wb[:]), 0.0)


def _up(h, parts, wa, wb):
    sds = jax.ShapeDtypeStruct((N, D), jnp.float32)
    return pl.pallas_call(_up_body, out_shape=sds)(h, parts, wa, wb)


def _rs_body(zc, zv, wmc, wlc, wmv, wlv, ec, ev, oc, ov):
    lvc = jnp.clip(_dot(zc[:], wlc[:]), -5.0, 5.0)
    oc[:] = _dot(zc[:], wmc[:]) + jnp.exp(0.5 * lvc) * ec[:]
    lvv = jnp.clip(_dot(zv[:], wlv[:]), -5.0, 5.0)
    ov[:] = _dot(zv[:], wmv[:]) + jnp.exp(0.5 * lvv) * ev[:]


def _resample(zc, zv, wmc, wlc, wmv, wlv, ec, ev):
    sds = jax.ShapeDtypeStruct((N, D), jnp.float32)
    return pl.pallas_call(
        _rs_body, out_shape=(sds, sds))(zc, zv, wmc, wlc, wmv, wlv, ec, ev)


def _sub_t_body(h2, zs, idx, wn, o, t_o):
    K = idx.shape[0]
    rows = lax.broadcasted_iota(jnp.int32, (N, 128), 0)
    hit = jnp.zeros((N, 128), jnp.float32)
    for k in range(K):
        hit = hit + (rows == idx[k, :][None, :]).astype(jnp.float32)
    mask = jnp.sum(hit, axis=1, keepdims=True) > 0.0
    hs = jnp.where(mask, zs[:], h2[:])
    o[:] = hs
    t_o[:] = _dot(hs, wn[:])


def _sub_t(h2, zs, idx, wn):
    sds = jax.ShapeDtypeStruct((N, D), jnp.float32)
    return pl.pallas_call(_sub_t_body, out_shape=(sds, sds))(h2, zs, idx, wn)


def _sub_body(h2, zs, idx, o):
    K = idx.shape[0]
    rows = lax.broadcasted_iota(jnp.int32, (N, 128), 0)
    hit = jnp.zeros((N, 128), jnp.float32)
    for k in range(K):
        hit = hit + (rows == idx[k, :][None, :]).astype(jnp.float32)
    mask = jnp.sum(hit, axis=1, keepdims=True) > 0.0
    o[:] = jnp.where(mask, zs[:], h2[:])


def _sub(h2, zs, idx):
    sds = jax.ShapeDtypeStruct((N, D), jnp.float32)
    return pl.pallas_call(_sub_body, out_shape=sds)(h2, zs, idx)


def kernel(x_constraints, x_variables, edge_index, edge_attr, community_idx,
           connected_vars_idx, eps_c, eps_v, W_emb_c, W_emb_v,
           Wm_vc, Wu_c, Wm_cv, Wu_v, W_mu_c, W_lv_c, W_mu_v, W_lv_v):
    npad = EROWS_PAD - EROWS
    src2d = edge_index[0].reshape(EROWS, 128)
    dst2d = edge_index[1].reshape(EROWS, 128)
    zpad = jnp.zeros((npad, 128), jnp.int32)
    jpad = jnp.full((npad, 128), JUNK_ROW, jnp.int32)
    src_g = jnp.concatenate([src2d, zpad])
    src_s = jnp.concatenate([src2d, jpad])
    dst_g = jnp.concatenate([dst2d, zpad])
    dst_s = jnp.concatenate([dst2d, jpad])
    ea_blk = edge_attr.T.reshape(4, EROWS, 128).transpose(1, 0, 2)
    ea_blk = jnp.concatenate([ea_blk, jnp.zeros((npad, 4, 128), jnp.float32)])
    idx_c = jnp.full((512,), -1, jnp.int32).at[:500].set(community_idx)
    idx_c = idx_c.reshape(4, 128)
    idx_v = jnp.full((2048,), -1, jnp.int32).at[:2000].set(connected_vars_idx)
    idx_v = idx_v.reshape(16, 128)

    h_c, h_v, t_v0, t_v1 = _emb(x_constraints, x_variables, W_emb_c, W_emb_v,
                                Wm_vc[0, :D], Wm_vc[1, :D])

    pc = _edge_pass(t_v0, dst_g, src_s, ea_blk, Wm_vc[0, D:])
    z_c, t_c0 = _up_t(h_c, pc, Wu_c[0, :D], Wu_c[0, D:], Wm_cv[0, :D])
    pv = _edge_pass(t_c0, src_g, dst_s, ea_blk, Wm_cv[0, D:])
    z_v = _up(h_v, pv, Wu_v[0, :D], Wu_v[0, D:])

    zs_c, zs_v = _resample(z_c, z_v, W_mu_c, W_lv_c, W_mu_v, W_lv_v,
                           eps_c, eps_v)

    pc2 = _edge_pass(t_v1, dst_g, src_s, ea_blk, Wm_vc[1, D:])
    h_c2, t_c1 = _up_t(h_c, pc2, Wu_c[1, :D], Wu_c[1, D:], Wm_cv[1, :D])
    pv2 = _edge_pass(t_c1, src_g, dst_s, ea_blk, Wm_cv[1, D:])
    h_v2 = _up(h_v, pv2, Wu_v[1, :D], Wu_v[1, D:])

    h_c2s = _sub(h_c2, zs_c, idx_c)
    h_v2s, t_v2 = _sub_t(h_v2, zs_v, idx_v, Wm_vc[2, :D])

    pc3 = _edge_pass(t_v2, dst_g, src_s, ea_blk, Wm_vc[2, D:])
    p_c, t_c2 = _up_t(h_c2s, pc3, Wu_c[2, :D], Wu_c[2, D:], Wm_cv[2, :D])
    pv3 = _edge_pass(t_c2, src_g, dst_s, ea_blk, Wm_cv[2, D:])
    p_v = _up(h_v2s, pv3, Wu_v[2, :D], Wu_v[2, D:])

    return jnp.concatenate([p_c, p_v], axis=0)


# speedup vs baseline: 1.0004x; 1.0004x over previous
import functools
import jax
import jax.numpy as jnp
from jax import lax
from jax.experimental import pallas as pl
from jax.experimental.pallas import tpu as pltpu
from jax.experimental.pallas import tpu_sc as plsc

N = 5000
D = 128
E = 320000
EROWS = E // 128
NWORK = 32
EROWS_PAD = 2560
ROWS_W = EROWS_PAD // NWORK
BLK = 2
IDX_CHUNK = 40
EA_CHUNK = 16
TILE_ROWS = 320
ACC_ROWS = 16 * TILE_ROWS
JUNK_ROW = ACC_ROWS - 1


def _edge_body(table, gidx, sidx, ea, w2, out,
               gidx_v, sidx_v, ea_v, w2_v, rows_v, acc, sem, sem2):
    c = lax.axis_index("c")
    s = lax.axis_index("s")
    wid = s * 2 + c
    base = wid * ROWS_W

    pltpu.sync_copy(w2, w2_v)
    w2v = [[w2_v[k, pl.ds(d * 16, 16)] for d in range(8)] for k in range(4)]

    pltpu.sync_copy(gidx.at[pl.ds(base * 128, IDX_CHUNK * 128)], gidx_v)
    pltpu.sync_copy(sidx.at[pl.ds(base * 128, IDX_CHUNK * 128)], sidx_v)
    pltpu.sync_copy(ea.at[pl.ds(base, EA_CHUNK)], ea_v)

    zrow = jnp.zeros((16,), jnp.float32)

    def zero_body(r, carry):
        for d in range(8):
            rows_v[r, pl.ds(d * 16, 16)] = zrow
        return carry

    lax.fori_loop(0, 128, zero_body, 0)
    base_acc = s * TILE_ROWS
    z128 = rows_v.at[pl.ds(0, 128)]
    pltpu.sync_copy(z128, acc.at[pl.ds(base_acc, 128)])
    pltpu.sync_copy(z128, acc.at[pl.ds(base_acc + 128, 128)])
    pltpu.sync_copy(rows_v.at[pl.ds(0, 64)], acc.at[pl.ds(base_acc + 256, 64)])
    plsc.subcore_barrier()

    dnums = lax.GatherDimensionNumbers(
        offset_dims=(), collapsed_slice_dims=(0,), start_index_map=(0,))

    def compute(ei, br):
        def grp(g, carry2):
            eak = [ea_v[ei, k, pl.ds(g * 16, 16)] for k in range(4)]
            for j in range(16):
                jidx = jnp.full((16, 1), j, jnp.int32)
                b = [lax.gather(eak[k], jidx, dnums, slice_sizes=(1,),
                                mode=lax.GatherScatterMode.PROMISE_IN_BOUNDS)
                     for k in range(4)]
                e = br * 128 + g * 16 + j
                for d in range(8):
                    r = rows_v[e, pl.ds(d * 16, 16)]
                    r = (r + b[0] * w2v[0][d] + b[1] * w2v[1][d]
                         + b[2] * w2v[2][d] + b[3] * w2v[3][d])
                    rows_v[e, pl.ds(d * 16, 16)] = jnp.maximum(r, 0.0)
            return carry2

        lax.fori_loop(0, 8, grp, 0)

    def group(g, carry):
        r0 = BLK * g
        ic = lax.div(g, IDX_CHUNK // BLK)
        ec = lax.div(g, EA_CHUNK // BLK)

        @pl.when(jnp.logical_and(g > 0, lax.rem(g, IDX_CHUNK // BLK) == 0))
        def _():
            pltpu.sync_copy(
                gidx.at[pl.ds((base + ic * IDX_CHUNK) * 128, IDX_CHUNK * 128)],
                gidx_v)
            pltpu.sync_copy(
                sidx.at[pl.ds((base + ic * IDX_CHUNK) * 128, IDX_CHUNK * 128)],
                sidx_v)

        @pl.when(jnp.logical_and(g > 0, lax.rem(g, EA_CHUNK // BLK) == 0))
        def _():
            pltpu.sync_copy(ea.at[pl.ds(base + ec * EA_CHUNK, EA_CHUNK)], ea_v)

        lr = lax.rem(r0, IDX_CHUNK)
        pltpu.async_copy(table.at[gidx_v.at[pl.ds(lr * 128, BLK * 128)]],
                         rows_v, sem).wait()
        for br in range(BLK):
            compute(lax.rem(r0 + br, EA_CHUNK), br)
        pltpu.sync_copy(rows_v, acc.at[sidx_v.at[pl.ds(lr * 128, BLK * 128)]],
                        add=True)
        return carry

    lax.fori_loop(0, ROWS_W // BLK, group, 0)
    plsc.subcore_barrier()

    for off, nr in ((0, 128), (128, 128), (256, 64)):
        pltpu.sync_copy(acc.at[pl.ds(base_acc + off, nr)],
                        rows_v.at[pl.ds(0, nr)])
        pltpu.sync_copy(rows_v.at[pl.ds(0, nr)],
                        out.at[c, pl.ds(base_acc + off, nr)])


def _edge_pass(table, gidx, sidx, ea, w2):
    mesh = plsc.VectorSubcoreMesh(core_axis_name="c", subcore_axis_name="s")
    f = pl.kernel(
        _edge_body,
        mesh=mesh,
        out_type=jax.ShapeDtypeStruct((2, ACC_ROWS, D), jnp.float32),
        scratch_types=[
            pltpu.VMEM((IDX_CHUNK * 128,), jnp.int32),
            pltpu.VMEM((IDX_CHUNK * 128,), jnp.int32),
            pltpu.VMEM((EA_CHUNK, 4, 128), jnp.float32),
            pltpu.VMEM((4, 128), jnp.float32),
            pltpu.VMEM((BLK * 128, 128), jnp.float32),
            pltpu.VMEM_SHARED((ACC_ROWS, 128), jnp.float32),
            pltpu.SemaphoreType.DMA,
            pltpu.SemaphoreType.DMA,
        ],
    )
    return f(table, gidx, sidx, ea, w2)


def _dot(a, b):
    return jnp.dot(a, b, preferred_element_type=jnp.float32)


def _emb_body(xc, xv, wc, wv, w0, w1, hc_o, hv_o, t0_o, t1_o):
    hc = jnp.maximum(_dot(xc[:], wc[:]), 0.0)
    hv = jnp.maximum(_dot(xv[:], wv[:]), 0.0)
    hc_o[:] = hc
    hv_o[:] = hv
    t0_o[:] = _dot(hv, w0[:])
    t1_o[:] = _dot(hv, w1[:])


def _emb(xc, xv, wc, wv, w0, w1):
    sds = jax.ShapeDtypeStruct((N, D), jnp.float32)
    return pl.pallas_call(
        _emb_body, out_shape=(sds, sds, sds, sds))(xc, xv, wc, wv, w0, w1)


def _up_t_body(h, parts, wa, wb, wn, h_o, t_o):
    agg = parts[0, :N, :] + parts[1, :N, :]
    hn = jnp.maximum(_dot(h[:], wa[:]) + _dot(agg, wb[:]), 0.0)
    h_o[:] = hn
    t_o[:] = _dot(hn, wn[:])


def _up_t(h, parts, wa, wb, wn):
    sds = jax.ShapeDtypeStruct((N, D), jnp.float32)
    return pl.pallas_call(
        _up_t_body, out_shape=(sds, sds))(h, parts, wa, wb, wn)


def _up_body(h, parts, wa, wb, h_o):
    agg = parts[0, :N, :] + parts[1, :N, :]
    h_o[:] = jnp.maximum(_dot(h[:], wa[:]) + _dot(agg, wb[:]), 0.0)


def _up(h, parts, wa, wb):
    sds = jax.ShapeDtypeStruct((N, D), jnp.float32)
    return pl.pallas_call(_up_body, out_shape=sds)(h, parts, wa, wb)


def _rs_body(zc, zv, wmc, wlc, wmv, wlv, ec, ev, oc, ov):
    lvc = jnp.clip(_dot(zc[:], wlc[:]), -5.0, 5.0)
    oc[:] = _dot(zc[:], wmc[:]) + jnp.exp(0.5 * lvc) * ec[:]
    lvv = jnp.clip(_dot(zv[:], wlv[:]), -5.0, 5.0)
    ov[:] = _dot(zv[:], wmv[:]) + jnp.exp(0.5 * lvv) * ev[:]


def _resample(zc, zv, wmc, wlc, wmv, wlv, ec, ev):
    sds = jax.ShapeDtypeStruct((N, D), jnp.float32)
    return pl.pallas_call(
        _rs_body, out_shape=(sds, sds))(zc, zv, wmc, wlc, wmv, wlv, ec, ev)


def _sub_t_body(h2, zs, idx, wn, o, t_o):
    K = idx.shape[0]
    rows = lax.broadcasted_iota(jnp.int32, (N, 128), 0)
    hit = jnp.zeros((N, 128), jnp.float32)
    for k in range(K):
        hit = hit + (rows == idx[k, :][None, :]).astype(jnp.float32)
    mask = jnp.sum(hit, axis=1, keepdims=True) > 0.0
    hs = jnp.where(mask, zs[:], h2[:])
    o[:] = hs
    t_o[:] = _dot(hs, wn[:])


def _sub_t(h2, zs, idx, wn):
    sds = jax.ShapeDtypeStruct((N, D), jnp.float32)
    return pl.pallas_call(_sub_t_body, out_shape=(sds, sds))(h2, zs, idx, wn)


def _sub_body(h2, zs, idx, o):
    K = idx.shape[0]
    rows = lax.broadcasted_iota(jnp.int32, (N, 128), 0)
    hit = jnp.zeros((N, 128), jnp.float32)
    for k in range(K):
        hit = hit + (rows == idx[k, :][None, :]).astype(jnp.float32)
    mask = jnp.sum(hit, axis=1, keepdims=True) > 0.0
    o[:] = jnp.where(mask, zs[:], h2[:])


def _sub(h2, zs, idx):
    sds = jax.ShapeDtypeStruct((N, D), jnp.float32)
    return pl.pallas_call(_sub_body, out_shape=sds)(h2, zs, idx)


def kernel(x_constraints, x_variables, edge_index, edge_attr, community_idx,
           connected_vars_idx, eps_c, eps_v, W_emb_c, W_emb_v,
           Wm_vc, Wu_c, Wm_cv, Wu_v, W_mu_c, W_lv_c, W_mu_v, W_lv_v):
    npad = EROWS_PAD - EROWS
    src2d = edge_index[0].reshape(EROWS, 128)
    dst2d = edge_index[1].reshape(EROWS, 128)
    zpad = jnp.zeros((npad, 128), jnp.int32)
    jpad = jnp.full((npad, 128), JUNK_ROW, jnp.int32)
    src_g = jnp.concatenate([src2d, zpad]).reshape(-1)
    src_s = jnp.concatenate([src2d, jpad]).reshape(-1)
    dst_g = jnp.concatenate([dst2d, zpad]).reshape(-1)
    dst_s = jnp.concatenate([dst2d, jpad]).reshape(-1)
    ea_blk = edge_attr.T.reshape(4, EROWS, 128).transpose(1, 0, 2)
    ea_blk = jnp.concatenate([ea_blk, jnp.zeros((npad, 4, 128), jnp.float32)])
    idx_c = jnp.full((512,), -1, jnp.int32).at[:500].set(community_idx)
    idx_c = idx_c.reshape(4, 128)
    idx_v = jnp.full((2048,), -1, jnp.int32).at[:2000].set(connected_vars_idx)
    idx_v = idx_v.reshape(16, 128)

    h_c, h_v, t_v0, t_v1 = _emb(x_constraints, x_variables, W_emb_c, W_emb_v,
                                Wm_vc[0, :D], Wm_vc[1, :D])

    pc = _edge_pass(t_v0, dst_g, src_s, ea_blk, Wm_vc[0, D:])
    z_c, t_c0 = _up_t(h_c, pc, Wu_c[0, :D], Wu_c[0, D:], Wm_cv[0, :D])
    pv = _edge_pass(t_c0, src_g, dst_s, ea_blk, Wm_cv[0, D:])
    z_v = _up(h_v, pv, Wu_v[0, :D], Wu_v[0, D:])

    zs_c, zs_v = _resample(z_c, z_v, W_mu_c, W_lv_c, W_mu_v, W_lv_v,
                           eps_c, eps_v)

    pc2 = _edge_pass(t_v1, dst_g, src_s, ea_blk, Wm_vc[1, D:])
    h_c2, t_c1 = _up_t(h_c, pc2, Wu_c[1, :D], Wu_c[1, D:], Wm_cv[1, :D])
    pv2 = _edge_pass(t_c1, src_g, dst_s, ea_blk, Wm_cv[1, D:])
    h_v2 = _up(h_v, pv2, Wu_v[1, :D], Wu_v[1, D:])

    h_c2s = _sub(h_c2, zs_c, idx_c)
    h_v2s, t_v2 = _sub_t(h_v2, zs_v, idx_v, Wm_vc[2, :D])

    pc3 = _edge_pass(t_v2, dst_g, src_s, ea_blk, Wm_vc[2, D:])
    p_c, t_c2 = _up_t(h_c2s, pc3, Wu_c[2, :D], Wu_c[2, D:], Wm_cv[2, :D])
    pv3 = _edge_pass(t_c2, src_g, dst_s, ea_blk, Wm_cv[2, D:])
    p_v = _up(h_v2s, pv3, Wu_v[2, :D], Wu_v[2, D:])

    return jnp.concatenate([p_c, p_v], axis=0)


# speedup vs baseline: 1.7215x; 1.7208x over previous
import jax
import jax.numpy as jnp
from jax import lax
from jax.experimental import pallas as pl
from jax.experimental.pallas import tpu as pltpu
from jax.experimental.pallas import tpu_sc as plsc

N = 5000
D = 128
E = 320000
EROWS = E // 128
NWORK = 32
ROWS_W = EROWS // NWORK
ROWS_REM = EROWS - NWORK * ROWS_W
TILE_ROWS = 320
ACC_ROWS = 16 * TILE_ROWS


def _edge_body(table, pk, w2, out, pk_v, w2_v, rows_v, acc, sem):
    c = lax.axis_index("c")
    s = lax.axis_index("s")
    wid = s * 2 + c

    pltpu.sync_copy(w2, w2_v)
    w2v = [[w2_v[k, pl.ds(d * 16, 16)] for d in range(8)] for k in range(4)]

    zrow = jnp.zeros((16,), jnp.float32)

    def zero_body(r, carry):
        for d in range(8):
            rows_v[r, pl.ds(d * 16, 16)] = zrow
        return carry

    lax.fori_loop(0, 128, zero_body, 0)
    base_acc = s * TILE_ROWS
    pltpu.sync_copy(rows_v, acc.at[pl.ds(base_acc, 128)])
    pltpu.sync_copy(rows_v, acc.at[pl.ds(base_acc + 128, 128)])
    pltpu.sync_copy(rows_v.at[pl.ds(0, 64)],
                    acc.at[pl.ds(base_acc + 256, 64)])
    plsc.subcore_barrier()

    nrows = ROWS_W + jnp.where(wid < ROWS_REM, 1, 0)
    base = wid * ROWS_W + jnp.minimum(wid, ROWS_REM)

    dnums = lax.GatherDimensionNumbers(
        offset_dims=(), collapsed_slice_dims=(0,), start_index_map=(0,))

    def blk(i, carry):
        row = base + i
        pltpu.sync_copy(pk.at[row], pk_v)
        pltpu.async_copy(table.at[pk_v.at[0]], rows_v, sem).wait()

        def grp(g, carry2):
            eak = [lax.bitcast_convert_type(
                        pk_v[2 + k, pl.ds(g * 16, 16)], jnp.float32)
                   for k in range(4)]
            for j in range(16):
                jidx = jnp.full((16, 1), j, jnp.int32)
                b = [lax.gather(eak[k], jidx, dnums, slice_sizes=(1,),
                                mode=lax.GatherScatterMode.PROMISE_IN_BOUNDS)
                     for k in range(4)]
                e = g * 16 + j
                for d in range(8):
                    r = rows_v[e, pl.ds(d * 16, 16)]
                    r = (r + b[0] * w2v[0][d] + b[1] * w2v[1][d]
                         + b[2] * w2v[2][d] + b[3] * w2v[3][d])
                    rows_v[e, pl.ds(d * 16, 16)] = jnp.maximum(r, 0.0)
            return carry2

        lax.fori_loop(0, 8, grp, 0)
        pltpu.sync_copy(rows_v, acc.at[pk_v.at[1]], add=True)
        return carry

    lax.fori_loop(0, nrows, blk, 0)
    plsc.subcore_barrier()

    for off, nr in ((0, 128), (128, 128), (256, 64)):
        pltpu.sync_copy(acc.at[pl.ds(base_acc + off, nr)],
                        rows_v.at[pl.ds(0, nr)])
        pltpu.sync_copy(rows_v.at[pl.ds(0, nr)],
                        out.at[c, pl.ds(base_acc + off, nr)])


def _edge_pass(table, pk, w2):
    mesh = plsc.VectorSubcoreMesh(core_axis_name="c", subcore_axis_name="s")
    f = pl.kernel(
        _edge_body,
        mesh=mesh,
        out_type=jax.ShapeDtypeStruct((2, ACC_ROWS, D), jnp.float32),
        scratch_types=[
            pltpu.VMEM((6, 128), jnp.int32),
            pltpu.VMEM((4, 128), jnp.float32),
            pltpu.VMEM((128, 128), jnp.float32),
            pltpu.VMEM_SHARED((ACC_ROWS, 128), jnp.float32),
            pltpu.SemaphoreType.DMA,
        ],
    )
    return f(table, pk, w2)


def _dot(a, b):
    return jnp.dot(a, b, preferred_element_type=jnp.float32)


def _emb_body(xc, xv, wc, wv, w0, w1, hc_o, hv_o, t0_o, t1_o):
    hc = jnp.maximum(_dot(xc[:], wc[:]), 0.0)
    hv = jnp.maximum(_dot(xv[:], wv[:]), 0.0)
    hc_o[:] = hc
    hv_o[:] = hv
    t0_o[:] = _dot(hv, w0[:])
    t1_o[:] = _dot(hv, w1[:])


def _emb(xc, xv, wc, wv, w0, w1):
    sds = jax.ShapeDtypeStruct((N, D), jnp.float32)
    return pl.pallas_call(
        _emb_body, out_shape=(sds, sds, sds, sds))(xc, xv, wc, wv, w0, w1)


def _up_t_body(h, parts, wa, wb, wn, h_o, t_o):
    agg = parts[0] + parts[1]
    hn = jnp.maximum(_dot(h[:], wa[:]) + _dot(agg, wb[:]), 0.0)
    h_o[:] = hn
    t_o[:] = _dot(hn, wn[:])


def _up_t(h, parts, wa, wb, wn):
    sds = jax.ShapeDtypeStruct((N, D), jnp.float32)
    return pl.pallas_call(
        _up_t_body, out_shape=(sds, sds))(h, parts, wa, wb, wn)


def _up_body(h, parts, wa, wb, h_o):
    agg = parts[0] + parts[1]
    h_o[:] = jnp.maximum(_dot(h[:], wa[:]) + _dot(agg, wb[:]), 0.0)


def _up(h, parts, wa, wb):
    sds = jax.ShapeDtypeStruct((N, D), jnp.float32)
    return pl.pallas_call(_up_body, out_shape=sds)(h, parts, wa, wb)


def _rs_body(zc, zv, wmc, wlc, wmv, wlv, ec, ev, oc, ov):
    lvc = jnp.clip(_dot(zc[:], wlc[:]), -5.0, 5.0)
    oc[:] = _dot(zc[:], wmc[:]) + jnp.exp(0.5 * lvc) * ec[:]
    lvv = jnp.clip(_dot(zv[:], wlv[:]), -5.0, 5.0)
    ov[:] = _dot(zv[:], wmv[:]) + jnp.exp(0.5 * lvv) * ev[:]


def _resample(zc, zv, wmc, wlc, wmv, wlv, ec, ev):
    sds = jax.ShapeDtypeStruct((N, D), jnp.float32)
    return pl.pallas_call(
        _rs_body, out_shape=(sds, sds))(zc, zv, wmc, wlc, wmv, wlv, ec, ev)


def _sub_t_body(h2, zs, idx, wn, o, t_o):
    K = idx.shape[0]
    rows = lax.broadcasted_iota(jnp.int32, (N, 128), 0)
    hit = jnp.zeros((N, 128), jnp.float32)
    for k in range(K):
        hit = hit + (rows == idx[k, :][None, :]).astype(jnp.float32)
    mask = jnp.sum(hit, axis=1, keepdims=True) > 0.0
    hs = jnp.where(mask, zs[:], h2[:])
    o[:] = hs
    t_o[:] = _dot(hs, wn[:])


def _sub_t(h2, zs, idx, wn):
    sds = jax.ShapeDtypeStruct((N, D), jnp.float32)
    return pl.pallas_call(_sub_t_body, out_shape=(sds, sds))(h2, zs, idx, wn)


def _sub_body(h2, zs, idx, o):
    K = idx.shape[0]
    rows = lax.broadcasted_iota(jnp.int32, (N, 128), 0)
    hit = jnp.zeros((N, 128), jnp.float32)
    for k in range(K):
        hit = hit + (rows == idx[k, :][None, :]).astype(jnp.float32)
    mask = jnp.sum(hit, axis=1, keepdims=True) > 0.0
    o[:] = jnp.where(mask, zs[:], h2[:])


def _sub(h2, zs, idx):
    sds = jax.ShapeDtypeStruct((N, D), jnp.float32)
    return pl.pallas_call(_sub_body, out_shape=sds)(h2, zs, idx)


def kernel(x_constraints, x_variables, edge_index, edge_attr, community_idx,
           connected_vars_idx, eps_c, eps_v, W_emb_c, W_emb_v,
           Wm_vc, Wu_c, Wm_cv, Wu_v, W_mu_c, W_lv_c, W_mu_v, W_lv_v):
    src2d = edge_index[0].reshape(EROWS, 1, 128)
    dst2d = edge_index[1].reshape(EROWS, 1, 128)
    ea_i = lax.bitcast_convert_type(
        edge_attr.T.reshape(4, EROWS, 128).transpose(1, 0, 2), jnp.int32)
    pk_vc = jnp.concatenate([dst2d, src2d, ea_i], axis=1)
    pk_cv = jnp.concatenate([src2d, dst2d, ea_i], axis=1)
    idx_c = jnp.full((512,), -1, jnp.int32).at[:500].set(community_idx)
    idx_c = idx_c.reshape(4, 128)
    idx_v = jnp.full((2048,), -1, jnp.int32).at[:2000].set(connected_vars_idx)
    idx_v = idx_v.reshape(16, 128)

    h_c, h_v, t_v0, t_v1 = _emb(x_constraints, x_variables, W_emb_c, W_emb_v,
                                Wm_vc[0, :D], Wm_vc[1, :D])

    pc = _edge_pass(t_v0, pk_vc, Wm_vc[0, D:])
    z_c, t_c0 = _up_t(h_c, pc[:, :N, :], Wu_c[0, :D], Wu_c[0, D:],
                      Wm_cv[0, :D])
    pv = _edge_pass(t_c0, pk_cv, Wm_cv[0, D:])
    z_v = _up(h_v, pv[:, :N, :], Wu_v[0, :D], Wu_v[0, D:])

    pc2 = _edge_pass(t_v1, pk_vc, Wm_vc[1, D:])
    h_c2, t_c1 = _up_t(h_c, pc2[:, :N, :], Wu_c[1, :D], Wu_c[1, D:],
                       Wm_cv[1, :D])
    pv2 = _edge_pass(t_c1, pk_cv, Wm_cv[1, D:])
    h_v2 = _up(h_v, pv2[:, :N, :], Wu_v[1, :D], Wu_v[1, D:])

    zs_c, zs_v = _resample(z_c, z_v, W_mu_c, W_lv_c, W_mu_v, W_lv_v,
                           eps_c, eps_v)

    h_c2s = _sub(h_c2, zs_c, idx_c)
    h_v2s, t_v2 = _sub_t(h_v2, zs_v, idx_v, Wm_vc[2, :D])

    pc3 = _edge_pass(t_v2, pk_vc, Wm_vc[2, D:])
    p_c, t_c2 = _up_t(h_c2s, pc3[:, :N, :], Wu_c[2, :D], Wu_c[2, D:],
                      Wm_cv[2, :D])
    pv3 = _edge_pass(t_c2, pk_cv, Wm_cv[2, D:])
    p_v = _up(h_v2s, pv3[:, :N, :], Wu_v[2, :D], Wu_v[2, D:])

    return jnp.concatenate([p_c, p_v], axis=0)


# speedup vs baseline: 1.9312x; 1.1218x over previous
import jax
import jax.numpy as jnp
from jax import lax
from jax.experimental import pallas as pl
from jax.experimental.pallas import tpu as pltpu
from jax.experimental.pallas import tpu_sc as plsc

N = 5000
D = 128
E = 320000
EROWS = E // 128
NWORK = 32
ROWS_W = EROWS // NWORK
ROWS_REM = EROWS - NWORK * ROWS_W
TILE_ROWS = 320
ACC_ROWS = 16 * TILE_ROWS


def _edge_body(table, pk, w2, out, pk_v, w2_v, rows_v, acc,
               sem, sem_p0, sem_p1):
    c = lax.axis_index("c")
    s = lax.axis_index("s")
    wid = s * 2 + c

    pltpu.sync_copy(w2, w2_v)
    w2v = [[w2_v[k, pl.ds(d * 16, 16)] for d in range(8)] for k in range(4)]

    zrow = jnp.zeros((16,), jnp.float32)

    def zero_body(r, carry):
        for d in range(8):
            rows_v[r, pl.ds(d * 16, 16)] = zrow
        return carry

    lax.fori_loop(0, 128, zero_body, 0)
    base_acc = s * TILE_ROWS
    pltpu.sync_copy(rows_v, acc.at[pl.ds(base_acc, 128)])
    pltpu.sync_copy(rows_v, acc.at[pl.ds(base_acc + 128, 128)])
    pltpu.sync_copy(rows_v.at[pl.ds(0, 64)],
                    acc.at[pl.ds(base_acc + 256, 64)])
    plsc.subcore_barrier()

    nrows = ROWS_W + jnp.where(wid < ROWS_REM, 1, 0)
    base = wid * ROWS_W + jnp.minimum(wid, ROWS_REM)

    dnums = lax.GatherDimensionNumbers(
        offset_dims=(), collapsed_slice_dims=(0,), start_index_map=(0,))

    def prefetch(row, slot, sem_g):
        pltpu.async_copy(pk.at[row], pk_v.at[slot], sem_g)

    def do_blk(i, row, pks, rows_s, sem_p, sem_g):
        pltpu.make_async_copy(pk.at[row], pks, sem_p).wait()
        pltpu.async_copy(table.at[pks.at[0]], rows_s, sem_g).wait()

        def grp(g, carry2):
            eak = [lax.bitcast_convert_type(
                        pks[2 + k, pl.ds(g * 16, 16)], jnp.float32)
                   for k in range(4)]
            for j in range(16):
                jidx = jnp.full((16, 1), j, jnp.int32)
                b = [lax.gather(eak[k], jidx, dnums, slice_sizes=(1,),
                                mode=lax.GatherScatterMode.PROMISE_IN_BOUNDS)
                     for k in range(4)]
                e = g * 16 + j
                for d in range(8):
                    r = rows_s[e, pl.ds(d * 16, 16)]
                    r = (r + b[0] * w2v[0][d] + b[1] * w2v[1][d]
                         + b[2] * w2v[2][d] + b[3] * w2v[3][d])
                    rows_s[e, pl.ds(d * 16, 16)] = jnp.maximum(r, 0.0)
            return carry2

        lax.fori_loop(0, 8, grp, 0)
        pltpu.sync_copy(rows_s, acc.at[pks.at[1]], add=True)

    def blk(i, carry):
        row = base + i
        par = lax.rem(i, 2)

        @pl.when(jnp.logical_and(par == 0, i + 1 < nrows))
        def _():
            prefetch(row + 1, 1, sem_p1)

        @pl.when(jnp.logical_and(par == 1, i + 1 < nrows))
        def _():
            prefetch(row + 1, 0, sem_p0)

        @pl.when(par == 0)
        def _():
            do_blk(i, row, pk_v.at[0], rows_v, sem_p0, sem)

        @pl.when(par == 1)
        def _():
            do_blk(i, row, pk_v.at[1], rows_v, sem_p1, sem)
        return carry

    prefetch(base, 0, sem_p0)
    lax.fori_loop(0, nrows, blk, 0)
    plsc.subcore_barrier()

    for off, nr in ((0, 128), (128, 128), (256, 64)):
        pltpu.sync_copy(acc.at[pl.ds(base_acc + off, nr)],
                        rows_v.at[pl.ds(0, nr)])
        pltpu.sync_copy(rows_v.at[pl.ds(0, nr)],
                        out.at[c, pl.ds(base_acc + off, nr)])


def _edge_pass(table, pk, w2):
    mesh = plsc.VectorSubcoreMesh(core_axis_name="c", subcore_axis_name="s")
    f = pl.kernel(
        _edge_body,
        mesh=mesh,
        out_type=jax.ShapeDtypeStruct((2, ACC_ROWS, D), jnp.float32),
        scratch_types=[
            pltpu.VMEM((2, 6, 128), jnp.int32),
            pltpu.VMEM((4, 128), jnp.float32),
            pltpu.VMEM((128, 128), jnp.float32),
            pltpu.VMEM_SHARED((ACC_ROWS, 128), jnp.float32),
            pltpu.SemaphoreType.DMA,
            pltpu.SemaphoreType.DMA,
            pltpu.SemaphoreType.DMA,
        ],
    )
    return f(table, pk, w2)


def _dot(a, b):
    return jnp.dot(a, b, preferred_element_type=jnp.float32)


def _emb_body(xc, xv, wc, wv, w0, w1, hc_o, hv_o, t0_o, t1_o):
    hc = jnp.maximum(_dot(xc[:], wc[:]), 0.0)
    hv = jnp.maximum(_dot(xv[:], wv[:]), 0.0)
    hc_o[:] = hc
    hv_o[:] = hv
    t0_o[:] = _dot(hv, w0[:])
    t1_o[:] = _dot(hv, w1[:])


def _emb(xc, xv, wc, wv, w0, w1):
    sds = jax.ShapeDtypeStruct((N, D), jnp.float32)
    return pl.pallas_call(
        _emb_body, out_shape=(sds, sds, sds, sds))(xc, xv, wc, wv, w0, w1)


def _up_t_body(h, parts, wa, wb, wn, h_o, t_o):
    agg = parts[0] + parts[1]
    hn = jnp.maximum(_dot(h[:], wa[:]) + _dot(agg, wb[:]), 0.0)
    h_o[:] = hn
    t_o[:] = _dot(hn, wn[:])


def _up_t(h, parts, wa, wb, wn):
    sds = jax.ShapeDtypeStruct((N, D), jnp.float32)
    return pl.pallas_call(
        _up_t_body, out_shape=(sds, sds))(h, parts, wa, wb, wn)


def _up_body(h, parts, wa, wb, h_o):
    agg = parts[0] + parts[1]
    h_o[:] = jnp.maximum(_dot(h[:], wa[:]) + _dot(agg, wb[:]), 0.0)


def _up(h, parts, wa, wb):
    sds = jax.ShapeDtypeStruct((N, D), jnp.float32)
    return pl.pallas_call(_up_body, out_shape=sds)(h, parts, wa, wb)


def _rs_body(zc, zv, wmc, wlc, wmv, wlv, ec, ev, oc, ov):
    lvc = jnp.clip(_dot(zc[:], wlc[:]), -5.0, 5.0)
    oc[:] = _dot(zc[:], wmc[:]) + jnp.exp(0.5 * lvc) * ec[:]
    lvv = jnp.clip(_dot(zv[:], wlv[:]), -5.0, 5.0)
    ov[:] = _dot(zv[:], wmv[:]) + jnp.exp(0.5 * lvv) * ev[:]


def _resample(zc, zv, wmc, wlc, wmv, wlv, ec, ev):
    sds = jax.ShapeDtypeStruct((N, D), jnp.float32)
    return pl.pallas_call(
        _rs_body, out_shape=(sds, sds))(zc, zv, wmc, wlc, wmv, wlv, ec, ev)


def _sub_t_body(h2, zs, idx, wn, o, t_o):
    K = idx.shape[0]
    rows = lax.broadcasted_iota(jnp.int32, (N, 128), 0)
    hit = jnp.zeros((N, 128), jnp.float32)
    for k in range(K):
        hit = hit + (rows == idx[k, :][None, :]).astype(jnp.float32)
    mask = jnp.sum(hit, axis=1, keepdims=True) > 0.0
    hs = jnp.where(mask, zs[:], h2[:])
    o[:] = hs
    t_o[:] = _dot(hs, wn[:])


def _sub_t(h2, zs, idx, wn):
    sds = jax.ShapeDtypeStruct((N, D), jnp.float32)
    return pl.pallas_call(_sub_t_body, out_shape=(sds, sds))(h2, zs, idx, wn)


def _sub_body(h2, zs, idx, o):
    K = idx.shape[0]
    rows = lax.broadcasted_iota(jnp.int32, (N, 128), 0)
    hit = jnp.zeros((N, 128), jnp.float32)
    for k in range(K):
        hit = hit + (rows == idx[k, :][None, :]).astype(jnp.float32)
    mask = jnp.sum(hit, axis=1, keepdims=True) > 0.0
    o[:] = jnp.where(mask, zs[:], h2[:])


def _sub(h2, zs, idx):
    sds = jax.ShapeDtypeStruct((N, D), jnp.float32)
    return pl.pallas_call(_sub_body, out_shape=sds)(h2, zs, idx)


def kernel(x_constraints, x_variables, edge_index, edge_attr, community_idx,
           connected_vars_idx, eps_c, eps_v, W_emb_c, W_emb_v,
           Wm_vc, Wu_c, Wm_cv, Wu_v, W_mu_c, W_lv_c, W_mu_v, W_lv_v):
    src2d = edge_index[0].reshape(EROWS, 1, 128)
    dst2d = edge_index[1].reshape(EROWS, 1, 128)
    ea_i = lax.bitcast_convert_type(
        edge_attr.T.reshape(4, EROWS, 128).transpose(1, 0, 2), jnp.int32)
    pk_vc = jnp.concatenate([dst2d, src2d, ea_i], axis=1)
    pk_cv = jnp.concatenate([src2d, dst2d, ea_i], axis=1)
    idx_c = jnp.full((512,), -1, jnp.int32).at[:500].set(community_idx)
    idx_c = idx_c.reshape(4, 128)
    idx_v = jnp.full((2048,), -1, jnp.int32).at[:2000].set(connected_vars_idx)
    idx_v = idx_v.reshape(16, 128)

    h_c, h_v, t_v0, t_v1 = _emb(x_constraints, x_variables, W_emb_c, W_emb_v,
                                Wm_vc[0, :D], Wm_vc[1, :D])

    pc = _edge_pass(t_v0, pk_vc, Wm_vc[0, D:])
    z_c, t_c0 = _up_t(h_c, pc[:, :N, :], Wu_c[0, :D], Wu_c[0, D:],
                      Wm_cv[0, :D])
    pv = _edge_pass(t_c0, pk_cv, Wm_cv[0, D:])
    z_v = _up(h_v, pv[:, :N, :], Wu_v[0, :D], Wu_v[0, D:])

    pc2 = _edge_pass(t_v1, pk_vc, Wm_vc[1, D:])
    h_c2, t_c1 = _up_t(h_c, pc2[:, :N, :], Wu_c[1, :D], Wu_c[1, D:],
                       Wm_cv[1, :D])
    pv2 = _edge_pass(t_c1, pk_cv, Wm_cv[1, D:])
    h_v2 = _up(h_v, pv2[:, :N, :], Wu_v[1, :D], Wu_v[1, D:])

    zs_c, zs_v = _resample(z_c, z_v, W_mu_c, W_lv_c, W_mu_v, W_lv_v,
                           eps_c, eps_v)

    h_c2s = _sub(h_c2, zs_c, idx_c)
    h_v2s, t_v2 = _sub_t(h_v2, zs_v, idx_v, Wm_vc[2, :D])

    pc3 = _edge_pass(t_v2, pk_vc, Wm_vc[2, D:])
    p_c, t_c2 = _up_t(h_c2s, pc3[:, :N, :], Wu_c[2, :D], Wu_c[2, D:],
                      Wm_cv[2, :D])
    pv3 = _edge_pass(t_c2, pk_cv, Wm_cv[2, D:])
    p_v = _up(h_v2s, pv3[:, :N, :], Wu_v[2, :D], Wu_v[2, D:])

    return jnp.concatenate([p_c, p_v], axis=0)
